# baseline (speedup 1.0000x reference)
import jax
import jax.numpy as jnp
from jax.experimental import pallas as pl
from jax.experimental.pallas import tpu as pltpu

ALPHA, LIMIT = 1.702, 7.0


def _moe_body(sched_ref, x_ref, w1_ref, b1_ref, w2_ref, b2_ref, idx_ref,
              wgt_ref, out_ref):
    i = pl.program_id(0)
    n_steps = pl.num_programs(0)
    H = x_ref.shape[1]
    e = sched_ref[i]
    num_active = sched_ref[n_steps]

    @pl.when(i == 0)
    def _init():
        out_ref[...] = jnp.zeros_like(out_ref)

    @pl.when(i < num_active)
    def _compute():
        idx = idx_ref[0]
        wgt = wgt_ref[0].astype(jnp.float32)
        c = jnp.sum(jnp.where(idx == e, wgt, 0.0), axis=1, keepdims=True)

        x = x_ref[...]
        x_glu = jax.lax.dot_general(
            x, w1_ref[0, :, :H], (((1,), (1,)), ((), ())),
            preferred_element_type=jnp.float32)
        x_lin = jax.lax.dot_general(
            x, w1_ref[0, :, H:], (((1,), (1,)), ((), ())),
            preferred_element_type=jnp.float32)
        b1 = b1_ref[0].astype(jnp.float32)
        x_glu = x_glu + b1[0:1, :]
        x_lin = x_lin + b1[1:2, :]
        x_glu = x_glu.astype(jnp.bfloat16).astype(jnp.float32)
        x_lin = x_lin.astype(jnp.bfloat16).astype(jnp.float32)

        x_glu = jnp.minimum(x_glu, LIMIT)
        x_lin = jnp.clip(x_lin, -LIMIT, LIMIT)
        act = (x_glu * jax.nn.sigmoid(ALPHA * x_glu)) * (x_lin + 1.0)
        act = act.astype(jnp.bfloat16)

        t2 = jax.lax.dot_general(
            act, w2_ref[0], (((1,), (1,)), ((), ())),
            preferred_element_type=jnp.float32)
        t2 = t2 + b2_ref[0].astype(jnp.float32)

        out_ref[...] += t2 * c


def kernel(x, expert_weights, mlp1_weight, mlp1_bias, mlp2_weight, mlp2_bias,
           expert_indices):
    T, H = x.shape
    E, two_i, _ = mlp1_weight.shape
    K = expert_indices.shape[1]
    I = two_i // 2

    w1v = mlp1_weight.reshape(E, I, 2 * H)
    b1v = mlp1_bias.reshape(E, I, 2).transpose(0, 2, 1)
    b2r = mlp2_bias[:, None, :]
    idx32 = expert_indices.astype(jnp.int32)
    idx3 = idx32[None]
    wgt3 = expert_weights[None]

    counts = jnp.zeros((E,), jnp.int32).at[idx32.reshape(-1)].add(1)
    active = counts > 0
    na = jnp.sum(active.astype(jnp.int32))
    order = jnp.argsort(jnp.logical_not(active), stable=True).astype(jnp.int32)
    last_active = order[jnp.maximum(na - 1, 0)]
    steps = jnp.arange(E, dtype=jnp.int32)
    eids = jnp.where(steps < na, order, last_active)
    sched = jnp.concatenate([eids, na[None]])

    grid_spec = pltpu.PrefetchScalarGridSpec(
        num_scalar_prefetch=1,
        grid=(E,),
        in_specs=[
            pl.BlockSpec((T, H), lambda i, s: (0, 0)),
            pl.BlockSpec((1, I, 2 * H), lambda i, s: (s[i], 0, 0)),
            pl.BlockSpec((1, 2, I), lambda i, s: (s[i], 0, 0)),
            pl.BlockSpec((1, H, I), lambda i, s: (s[i], 0, 0)),
            pl.BlockSpec((1, 1, H), lambda i, s: (s[i], 0, 0)),
            pl.BlockSpec((1, T, K), lambda i, s: (0, 0, 0)),
            pl.BlockSpec((1, T, K), lambda i, s: (0, 0, 0)),
        ],
        out_specs=pl.BlockSpec((T, H), lambda i, s: (0, 0)),
    )

    out = pl.pallas_call(
        _moe_body,
        grid_spec=grid_spec,
        out_shape=jax.ShapeDtypeStruct((T, H), jnp.float32),
    )(sched, x, w1v, b1v, mlp2_weight, b2r, idx3, wgt3)

    return out.astype(x.dtype)


# speedup vs baseline: 1.0039x; 1.0039x over previous
import jax
import jax.numpy as jnp
from jax.experimental import pallas as pl
from jax.experimental.pallas import tpu as pltpu

ALPHA, LIMIT = 1.702, 7.0


def _moe_body(sched_ref, x_ref, w1_hbm, b1_hbm, w2_hbm, b2_hbm, idx_ref,
              wgt_ref, out_ref,
              w1_buf, b1_buf, w2_buf, b2_buf, sems):
    i = pl.program_id(0)
    n_steps = pl.num_programs(0)
    H = x_ref.shape[1]
    num_active = sched_ref[n_steps]

    def start_fetch(step, slot):
        e = sched_ref[step]
        pltpu.make_async_copy(w1_hbm.at[e], w1_buf.at[slot], sems.at[slot, 0]).start()
        pltpu.make_async_copy(b1_hbm.at[e], b1_buf.at[slot], sems.at[slot, 1]).start()
        pltpu.make_async_copy(w2_hbm.at[e], w2_buf.at[slot], sems.at[slot, 2]).start()
        pltpu.make_async_copy(b2_hbm.at[e], b2_buf.at[slot], sems.at[slot, 3]).start()

    def wait_fetch(slot):
        pltpu.make_async_copy(w1_hbm.at[0], w1_buf.at[slot], sems.at[slot, 0]).wait()
        pltpu.make_async_copy(b1_hbm.at[0], b1_buf.at[slot], sems.at[slot, 1]).wait()
        pltpu.make_async_copy(w2_hbm.at[0], w2_buf.at[slot], sems.at[slot, 2]).wait()
        pltpu.make_async_copy(b2_hbm.at[0], b2_buf.at[slot], sems.at[slot, 3]).wait()

    @pl.when(i == 0)
    def _init():
        out_ref[...] = jnp.zeros_like(out_ref)
        start_fetch(0, 0)

    @pl.when(i + 1 < num_active)
    def _prefetch():
        start_fetch(i + 1, (i + 1) % 2)

    @pl.when(i < num_active)
    def _compute():
        slot = i % 2
        wait_fetch(slot)
        e = sched_ref[i]

        idx = idx_ref[0]
        wgt = wgt_ref[0].astype(jnp.float32)
        c = jnp.sum(jnp.where(idx == e, wgt, 0.0), axis=1, keepdims=True)

        x = x_ref[...]
        x_glu = jax.lax.dot_general(
            x, w1_buf[slot, :, :H], (((1,), (1,)), ((), ())),
            preferred_element_type=jnp.float32)
        x_lin = jax.lax.dot_general(
            x, w1_buf[slot, :, H:], (((1,), (1,)), ((), ())),
            preferred_element_type=jnp.float32)
        b1 = b1_buf[slot].astype(jnp.float32)
        x_glu = x_glu + b1[0:1, :]
        x_lin = x_lin + b1[1:2, :]
        x_glu = x_glu.astype(jnp.bfloat16).astype(jnp.float32)
        x_lin = x_lin.astype(jnp.bfloat16).astype(jnp.float32)

        x_glu = jnp.minimum(x_glu, LIMIT)
        x_lin = jnp.clip(x_lin, -LIMIT, LIMIT)
        act = (x_glu * jax.nn.sigmoid(ALPHA * x_glu)) * (x_lin + 1.0)
        act = act.astype(jnp.bfloat16)

        t2 = jax.lax.dot_general(
            act, w2_buf[slot], (((1,), (1,)), ((), ())),
            preferred_element_type=jnp.float32)
        t2 = t2 + b2_buf[slot].astype(jnp.float32)

        out_ref[...] += t2 * c


def kernel(x, expert_weights, mlp1_weight, mlp1_bias, mlp2_weight, mlp2_bias,
           expert_indices):
    T, H = x.shape
    E, two_i, _ = mlp1_weight.shape
    K = expert_indices.shape[1]
    I = two_i // 2

    w1v = mlp1_weight.reshape(E, I, 2 * H)
    b1v = mlp1_bias.reshape(E, I, 2).transpose(0, 2, 1)
    b2r = mlp2_bias[:, None, :]
    idx32 = expert_indices.astype(jnp.int32)
    idx3 = idx32[None]
    wgt3 = expert_weights[None]

    counts = jnp.zeros((E,), jnp.int32).at[idx32.reshape(-1)].add(1)
    active = counts > 0
    na = jnp.sum(active.astype(jnp.int32))
    order = jnp.argsort(jnp.logical_not(active), stable=True).astype(jnp.int32)
    sched = jnp.concatenate([order, na[None]])

    grid_spec = pltpu.PrefetchScalarGridSpec(
        num_scalar_prefetch=1,
        grid=(E,),
        in_specs=[
            pl.BlockSpec((T, H), lambda i, s: (0, 0)),
            pl.BlockSpec(memory_space=pltpu.MemorySpace.HBM),
            pl.BlockSpec(memory_space=pltpu.MemorySpace.HBM),
            pl.BlockSpec(memory_space=pltpu.MemorySpace.HBM),
            pl.BlockSpec(memory_space=pltpu.MemorySpace.HBM),
            pl.BlockSpec((1, T, K), lambda i, s: (0, 0, 0)),
            pl.BlockSpec((1, T, K), lambda i, s: (0, 0, 0)),
        ],
        out_specs=pl.BlockSpec((T, H), lambda i, s: (0, 0)),
        scratch_shapes=[
            pltpu.VMEM((2, I, 2 * H), jnp.bfloat16),
            pltpu.VMEM((2, 2, I), jnp.bfloat16),
            pltpu.VMEM((2, H, I), jnp.bfloat16),
            pltpu.VMEM((2, 1, H), jnp.bfloat16),
            pltpu.SemaphoreType.DMA((2, 4)),
        ],
    )

    out = pl.pallas_call(
        _moe_body,
        grid_spec=grid_spec,
        out_shape=jax.ShapeDtypeStruct((T, H), jnp.float32),
        compiler_params=pltpu.CompilerParams(
            dimension_semantics=("arbitrary",)),
    )(sched, x, w1v, b1v, mlp2_weight, b2r, idx3, wgt3)

    return out.astype(x.dtype)


# speedup vs baseline: 1.0418x; 1.0377x over previous
import jax
import jax.numpy as jnp
from jax.experimental import pallas as pl
from jax.experimental.pallas import tpu as pltpu

ALPHA, LIMIT = 1.702, 7.0


def _moe_body(sched_ref, x_ref, w1_hbm, b1_ref, w2_hbm, b2_ref, idx_ref,
              wgt_ref, out_ref,
              w1_buf, w2_buf, sems):
    i = pl.program_id(0)
    n_steps = pl.num_programs(0)
    H = x_ref.shape[1]
    num_active = sched_ref[n_steps]
    NBUF = 4

    def start_fetch(step, slot):
        e = sched_ref[step]
        pltpu.make_async_copy(w1_hbm.at[e], w1_buf.at[slot], sems.at[slot, 0]).start()
        pltpu.make_async_copy(w2_hbm.at[e], w2_buf.at[slot], sems.at[slot, 1]).start()

    def wait_fetch(slot):
        pltpu.make_async_copy(w1_hbm.at[0], w1_buf.at[slot], sems.at[slot, 0]).wait()
        pltpu.make_async_copy(w2_hbm.at[0], w2_buf.at[slot], sems.at[slot, 1]).wait()

    @pl.when(i == 0)
    def _init():
        out_ref[...] = jnp.zeros_like(out_ref)
        start_fetch(0, 0)
        for j in range(1, NBUF - 1):
            @pl.when(j < num_active)
            def _p(j=j):
                start_fetch(j, j)

    @pl.when(i + NBUF - 1 < num_active)
    def _prefetch():
        start_fetch(i + NBUF - 1, (i + NBUF - 1) % NBUF)

    @pl.when(i < num_active)
    def _compute():
        slot = i % NBUF
        wait_fetch(slot)
        e = sched_ref[i]

        idx = idx_ref[0]
        wgt = wgt_ref[0].astype(jnp.float32)
        c = jnp.sum(jnp.where(idx == e, wgt, 0.0), axis=1, keepdims=True)

        x = x_ref[...]
        x_glu = jax.lax.dot_general(
            x, w1_buf[slot, :, :H], (((1,), (1,)), ((), ())),
            preferred_element_type=jnp.float32)
        x_lin = jax.lax.dot_general(
            x, w1_buf[slot, :, H:], (((1,), (1,)), ((), ())),
            preferred_element_type=jnp.float32)
        b1 = b1_ref[e].astype(jnp.float32)
        x_glu = x_glu + b1[0:1, :]
        x_lin = x_lin + b1[1:2, :]
        x_glu = x_glu.astype(jnp.bfloat16).astype(jnp.float32)
        x_lin = x_lin.astype(jnp.bfloat16).astype(jnp.float32)

        x_glu = jnp.minimum(x_glu, LIMIT)
        x_lin = jnp.clip(x_lin, -LIMIT, LIMIT)
        act = (x_glu * jax.nn.sigmoid(ALPHA * x_glu)) * (x_lin + 1.0)
        act = act.astype(jnp.bfloat16)

        t2 = jax.lax.dot_general(
            act, w2_buf[slot], (((1,), (1,)), ((), ())),
            preferred_element_type=jnp.float32)
        t2 = t2 + b2_ref[e].astype(jnp.float32)

        out_ref[...] += t2 * c


def kernel(x, expert_weights, mlp1_weight, mlp1_bias, mlp2_weight, mlp2_bias,
           expert_indices):
    T, H = x.shape
    E, two_i, _ = mlp1_weight.shape
    K = expert_indices.shape[1]
    I = two_i // 2

    w1v = mlp1_weight.reshape(E, I, 2 * H)
    b1v = mlp1_bias.reshape(E, I, 2).transpose(0, 2, 1)
    b2r = mlp2_bias[:, None, :]
    idx32 = expert_indices.astype(jnp.int32)
    idx3 = idx32[None]
    wgt3 = expert_weights[None]

    counts = jnp.zeros((E,), jnp.int32).at[idx32.reshape(-1)].add(1)
    active = counts > 0
    na = jnp.sum(active.astype(jnp.int32))
    order = jnp.argsort(jnp.logical_not(active), stable=True).astype(jnp.int32)
    sched = jnp.concatenate([order, na[None]])

    grid_spec = pltpu.PrefetchScalarGridSpec(
        num_scalar_prefetch=1,
        grid=(E,),
        in_specs=[
            pl.BlockSpec((T, H), lambda i, s: (0, 0)),
            pl.BlockSpec(memory_space=pltpu.MemorySpace.HBM),
            pl.BlockSpec((E, 2, I), lambda i, s: (0, 0, 0)),
            pl.BlockSpec(memory_space=pltpu.MemorySpace.HBM),
            pl.BlockSpec((E, 1, H), lambda i, s: (0, 0, 0)),
            pl.BlockSpec((1, T, K), lambda i, s: (0, 0, 0)),
            pl.BlockSpec((1, T, K), lambda i, s: (0, 0, 0)),
        ],
        out_specs=pl.BlockSpec((T, H), lambda i, s: (0, 0)),
        scratch_shapes=[
            pltpu.VMEM((4, I, 2 * H), jnp.bfloat16),
            pltpu.VMEM((4, H, I), jnp.bfloat16),
            pltpu.SemaphoreType.DMA((4, 2)),
        ],
    )

    out = pl.pallas_call(
        _moe_body,
        grid_spec=grid_spec,
        out_shape=jax.ShapeDtypeStruct((T, H), jnp.float32),
        compiler_params=pltpu.CompilerParams(
            dimension_semantics=("arbitrary",)),
    )(sched, x, w1v, b1v, mlp2_weight, b2r, idx3, wgt3)

    return out.astype(x.dtype)
